# Initial kernel scaffold; baseline (speedup 1.0000x reference)
import jax
import jax.numpy as jnp
from jax.experimental import pallas as pl


def kernel(tokens, text_emb_weight):
    raise NotImplementedError("write your pallas kernel here")


# speedup vs baseline: 1.4652x; 1.4652x over previous
import functools

import jax
import jax.numpy as jnp
from jax import lax
from jax.experimental import pallas as pl
from jax.experimental.pallas import tpu as pltpu
from jax.experimental.pallas import tpu_sc as plsc

LEN_TOK = 1000000
VOCAB = LEN_TOK + 1
EMB = 32
BATCH = 4096
SEQ = 200
B = BATCH * SEQ

NC = 2
NS = 16
NW = NC * NS
LANES = 16

B_PER_W = B // NW
CHUNK = 1600
NUM_CHUNKS = B_PER_W // CHUNK

assert B % (8 * NW) == 0
assert B_PER_W % CHUNK == 0 and CHUNK % 8 == 0


def _emb_body(tok_hbm, table_hbm, out_hbm, idx_v, rows_v, sem):
    wid = lax.axis_index("s") * NC + lax.axis_index("c")
    base = wid * B_PER_W

    def chunk_body(i, carry):
        off = base + i * CHUNK
        pltpu.sync_copy(tok_hbm.at[pl.ds(off, CHUNK)], idx_v)

        def shift(j, c):
            sl = pl.ds(j * LANES, LANES)
            t = idx_v[sl]
            idx_v[sl] = jnp.maximum(t + 1, 0)
            return c

        lax.fori_loop(0, CHUNK // LANES, shift, 0)

        pltpu.async_copy(table_hbm.at[idx_v], rows_v, sem).wait()
        pltpu.sync_copy(rows_v, out_hbm.at[pl.ds(off, CHUNK)])
        return carry

    lax.fori_loop(0, NUM_CHUNKS, chunk_body, 0)


@jax.jit
def kernel(tokens, text_emb_weight):
    flat = tokens.reshape(B)
    call = functools.partial(
        pl.kernel,
        mesh=plsc.VectorSubcoreMesh(core_axis_name="c", subcore_axis_name="s"),
        out_type=jax.ShapeDtypeStruct((B, EMB), jnp.float32),
        scratch_types=[
            pltpu.VMEM((CHUNK,), jnp.int32),
            pltpu.VMEM((CHUNK, EMB), jnp.float32),
            pltpu.SemaphoreType.DMA,
        ],
        compiler_params=pltpu.CompilerParams(use_tc_tiling_on_sc=False),
    )(_emb_body)
    out = call(flat, text_emb_weight)
    return out.reshape(BATCH, SEQ, EMB)


# speedup vs baseline: 1.9875x; 1.3565x over previous
import functools

import jax
import jax.numpy as jnp
from jax import lax
from jax.experimental import pallas as pl
from jax.experimental.pallas import tpu as pltpu
from jax.experimental.pallas import tpu_sc as plsc

LEN_TOK = 1000000
VOCAB = LEN_TOK + 1
EMB = 32
BATCH = 4096
SEQ = 200
B = BATCH * SEQ

NC = 2
NS = 16
NW = NC * NS
LANES = 16

B_PER_W = B // NW
CHUNK = 1600
NUM_CHUNKS = B_PER_W // CHUNK

assert B % (8 * NW) == 0
assert B_PER_W % CHUNK == 0 and CHUNK % 8 == 0


def _emb_body(tok_hbm, table_hbm, out_hbm, idx_v, rows_v, sem):
    wid = lax.axis_index("s") * NC + lax.axis_index("c")
    base = wid * B_PER_W

    def chunk_body(i, carry):
        off = base + i * CHUNK
        pltpu.sync_copy(tok_hbm.at[pl.ds(off, CHUNK)], idx_v)

        def shift(j, c):
            sl = pl.ds(j * LANES, LANES)
            t = idx_v[sl]
            idx_v[sl] = jnp.maximum(t + 1, 0)
            return c

        lax.fori_loop(0, CHUNK // LANES, shift, 0)

        pltpu.async_copy(table_hbm.at[idx_v], rows_v, sem).wait()
        pltpu.sync_copy(
            rows_v,
            out_hbm.at[pl.ds(off, CHUNK), pl.ds(0, EMB)],
        )
        return carry

    lax.fori_loop(0, NUM_CHUNKS, chunk_body, 0)


@jax.jit
def kernel(tokens, text_emb_weight):
    flat = tokens.reshape(B)
    call = functools.partial(
        pl.kernel,
        mesh=plsc.VectorSubcoreMesh(core_axis_name="c", subcore_axis_name="s"),
        out_type=jax.ShapeDtypeStruct((B, 128), jnp.float32),
        scratch_types=[
            pltpu.VMEM((CHUNK,), jnp.int32),
            pltpu.VMEM((CHUNK, EMB), jnp.float32),
            pltpu.SemaphoreType.DMA,
        ],
        compiler_params=pltpu.CompilerParams(use_tc_tiling_on_sc=False),
    )(_emb_body)
    out = call(flat, text_emb_weight)
    return out[:, :EMB].reshape(BATCH, SEQ, EMB)


# speedup vs baseline: 2.6940x; 1.3554x over previous
import functools

import jax
import jax.numpy as jnp
from jax import lax
from jax.experimental import layout as jexl
from jax.experimental import pallas as pl
from jax.experimental.pallas import tpu as pltpu
from jax.experimental.pallas import tpu_sc as plsc

LEN_TOK = 1000000
VOCAB = LEN_TOK + 1
EMB = 32
BATCH = 4096
SEQ = 200
B = BATCH * SEQ

NC = 2
NS = 16
NW = NC * NS
LANES = 16

B_PER_W = B // NW
CHUNK = 1600
NUM_CHUNKS = B_PER_W // CHUNK

assert B % (8 * NW) == 0
assert B_PER_W % CHUNK == 0 and CHUNK % 8 == 0


def _emb_body(tok_hbm, table_hbm, out_hbm, idx_v, rows_v, sem):
    wid = lax.axis_index("s") * NC + lax.axis_index("c")
    base = wid * B_PER_W

    def chunk_body(i, carry):
        off = base + i * CHUNK
        pltpu.sync_copy(tok_hbm.at[pl.ds(off, CHUNK)], idx_v)

        def shift(j, c):
            sl = pl.ds(j * LANES, LANES)
            t = idx_v[sl]
            idx_v[sl] = jnp.maximum(t + 1, 0)
            return c

        lax.fori_loop(0, CHUNK // LANES, shift, 0)

        pltpu.async_copy(table_hbm.at[idx_v], rows_v, sem).wait()
        pltpu.sync_copy(
            rows_v,
            out_hbm.at[pl.ds(off, CHUNK), pl.ds(0, EMB)],
        )
        return carry

    lax.fori_loop(0, NUM_CHUNKS, chunk_body, 0)


def _impl(tokens, text_emb_weight):
    flat = tokens.reshape(B)
    text_emb_weight = jexl.with_layout_constraint(
        text_emb_weight, jexl.Layout(major_to_minor=(0, 1))
    )
    call = functools.partial(
        pl.kernel,
        mesh=plsc.VectorSubcoreMesh(core_axis_name="c", subcore_axis_name="s"),
        out_type=jax.ShapeDtypeStruct((B, 128), jnp.float32),
        scratch_types=[
            pltpu.VMEM((CHUNK,), jnp.int32),
            pltpu.VMEM((CHUNK, EMB), jnp.float32),
            pltpu.SemaphoreType.DMA,
        ],
        compiler_params=pltpu.CompilerParams(use_tc_tiling_on_sc=False),
    )(_emb_body)
    out = call(flat, text_emb_weight)
    result = out[:, :EMB].reshape(BATCH, SEQ, EMB)
    return jexl.with_layout_constraint(
        result, jexl.Layout(major_to_minor=(0, 1, 2))
    )


@jax.jit
def kernel(tokens, text_emb_weight):
    return _impl(tokens, text_emb_weight)
